# Initial kernel scaffold; baseline (speedup 1.0000x reference)
import jax
import jax.numpy as jnp
from jax.experimental import pallas as pl


def kernel(x, edge_index, edge_weight, W_self, b_self, W_neigh, b_neigh):
    raise NotImplementedError("write your pallas kernel here")


# speedup vs baseline: 4.4680x; 4.4680x over previous
import functools

import jax
import jax.numpy as jnp
from jax import lax
from jax.experimental import pallas as pl
from jax.experimental.pallas import tpu as pltpu
from jax.experimental.pallas import tpu_sc as plsc

N = 10000
E = 320000
D = 128

NC = 2
NS = 16
NW = NC * NS
EDGES_PER_TILE = E // NW
K = 80
CHUNKS = EDGES_PER_TILE // K
ZCH = 80
NZCH = N // ZCH
ZITER = (NZCH + NS - 1) // NS

_mesh = plsc.VectorSubcoreMesh(core_axis_name="c", subcore_axis_name="s")


@functools.partial(
    pl.kernel,
    mesh=_mesh,
    out_type=jax.ShapeDtypeStruct((NC, N, D), jnp.float32),
    scratch_types=[
        pltpu.VMEM_SHARED((N, D), jnp.float32),
        pltpu.VMEM((K,), jnp.int32),
        pltpu.VMEM((K,), jnp.int32),
        pltpu.VMEM((K,), jnp.float32),
        pltpu.VMEM((K, D), jnp.float32),
        pltpu.SemaphoreType.DMA,
    ],
)
def _sc_segsum(x_hbm, src_hbm, dst_hbm, ew_hbm, out_hbm,
               acc, src_v, dst_v, ew_v, rows_v, sem):
    c = lax.axis_index("c")
    s = lax.axis_index("s")
    wid = c * NS + s

    zeros16 = jnp.zeros((16,), jnp.float32)

    def zrow(r, carry):
        for c8 in range(D // 16):
            rows_v[r, pl.ds(c8 * 16, 16)] = zeros16
        return carry

    lax.fori_loop(0, ZCH, zrow, 0)

    def zcopy(i, carry):
        z = s + i * NS

        @pl.when(z < NZCH)
        def _():
            pltpu.sync_copy(rows_v, acc.at[pl.ds(z * ZCH, ZCH)])

        return carry

    lax.fori_loop(0, ZITER, zcopy, 0)
    plsc.subcore_barrier()

    ebase = wid * EDGES_PER_TILE

    def chunk(j, carry):
        eb = ebase + j * K
        pltpu.sync_copy(src_hbm.at[pl.ds(eb, K)], src_v)
        pltpu.sync_copy(dst_hbm.at[pl.ds(eb, K)], dst_v)
        pltpu.sync_copy(ew_hbm.at[pl.ds(eb, K)], ew_v)
        pltpu.async_copy(x_hbm.at[src_v], rows_v, sem).wait()

        gdn = lax.GatherDimensionNumbers(
            offset_dims=(), collapsed_slice_dims=(0,), start_index_map=(0,))

        def egroup(g, icarry):
            w16 = ew_v[pl.ds(g * 16, 16)]
            for l in range(16):
                w = lax.gather(w16, jnp.full((16, 1), l, jnp.int32),
                               dimension_numbers=gdn, slice_sizes=(1,),
                               mode=lax.GatherScatterMode.PROMISE_IN_BOUNDS)
                k = g * 16 + l
                for c8 in range(D // 16):
                    sl = pl.ds(c8 * 16, 16)
                    rows_v[k, sl] = rows_v[k, sl] * w
            return icarry

        lax.fori_loop(0, K // 16, egroup, 0)
        pltpu.sync_copy(rows_v, acc.at[dst_v], add=True)
        return carry

    lax.fori_loop(0, CHUNKS, chunk, 0)
    plsc.subcore_barrier()

    def ocopy(i, carry):
        z = s + i * NS

        @pl.when(z < NZCH)
        def _():
            base = z * ZCH
            pltpu.sync_copy(acc.at[pl.ds(base, ZCH)],
                            out_hbm.at[c, pl.ds(base, ZCH)])

        return carry

    lax.fori_loop(0, ZITER, ocopy, 0)


BLK = 1000


def _tc_body(x_ref, p0_ref, p1_ref, ws_ref, wn_ref, bs_ref, bn_ref, o_ref):
    neigh = p0_ref[...] + p1_ref[...]
    dn = (((1,), (1,)), ((), ()))
    out = (lax.dot_general(x_ref[...], ws_ref[...], dn,
                           preferred_element_type=jnp.float32)
           + lax.dot_general(neigh, wn_ref[...], dn,
                             preferred_element_type=jnp.float32)
           + bs_ref[...] + bn_ref[...])
    o_ref[...] = jnp.maximum(out, 0.0)


@jax.jit
def _dense_tail(x, p0, p1, W_self, b_self, W_neigh, b_neigh):
    return pl.pallas_call(
        _tc_body,
        grid=(N // BLK,),
        in_specs=[
            pl.BlockSpec((BLK, D), lambda i: (i, 0)),
            pl.BlockSpec((BLK, D), lambda i: (i, 0)),
            pl.BlockSpec((BLK, D), lambda i: (i, 0)),
            pl.BlockSpec((D, D), lambda i: (0, 0)),
            pl.BlockSpec((D, D), lambda i: (0, 0)),
            pl.BlockSpec((1, D), lambda i: (0, 0)),
            pl.BlockSpec((1, D), lambda i: (0, 0)),
        ],
        out_specs=pl.BlockSpec((BLK, D), lambda i: (i, 0)),
        out_shape=jax.ShapeDtypeStruct((N, D), jnp.float32),
    )(x, p0, p1, W_self, W_neigh,
      b_self.reshape(1, D), b_neigh.reshape(1, D))


def kernel(x, edge_index, edge_weight, W_self, b_self, W_neigh, b_neigh):
    dst = edge_index[0].astype(jnp.int32)
    src = edge_index[1].astype(jnp.int32)
    partials = _sc_segsum(x, src, dst, edge_weight)
    return _dense_tail(x, partials[0], partials[1],
                       W_self, b_self, W_neigh, b_neigh)


# speedup vs baseline: 10.7823x; 2.4132x over previous
import functools

import jax
import jax.numpy as jnp
from jax import lax
from jax.experimental import pallas as pl
from jax.experimental.pallas import tpu as pltpu
from jax.experimental.pallas import tpu_sc as plsc

N = 10000
E = 320000
D = 128

NC = 2
NS = 16
NW = NC * NS
EPT = E // NW
K = 80
CHUNKS = EPT // K
ZCH = 80
NZCH = N // ZCH
ZITER = (NZCH + NS - 1) // NS

_mesh = plsc.VectorSubcoreMesh(core_axis_name="c", subcore_axis_name="s")

_GDN = lax.GatherDimensionNumbers(
    offset_dims=(), collapsed_slice_dims=(0,), start_index_map=(0,))


@functools.partial(
    pl.kernel,
    mesh=_mesh,
    out_type=jax.ShapeDtypeStruct((NC, N, D), jnp.float32),
    scratch_types=[
        pltpu.VMEM_SHARED((N, D), jnp.float32),
        pltpu.VMEM((EPT,), jnp.int32),
        pltpu.VMEM((EPT,), jnp.float32),
        pltpu.VMEM((K,), jnp.int32),
        pltpu.VMEM((K,), jnp.int32),
        pltpu.VMEM((K, D), jnp.float32),
        pltpu.VMEM((K, D), jnp.float32),
        pltpu.SemaphoreType.DMA,
        pltpu.SemaphoreType.DMA,
        pltpu.SemaphoreType.DMA,
        pltpu.SemaphoreType.DMA,
    ],
)
def _sc_segsum(x_hbm, src_hbm, dst_hbm, ew_hbm, out_hbm,
               acc, src_all, ew_all, dst0, dst1, rows0, rows1,
               gsem0, gsem1, dsem0, dsem1):
    c = lax.axis_index("c")
    s = lax.axis_index("s")
    wid = c * NS + s
    ebase = wid * EPT

    dst_b = (dst0, dst1)
    rows_b = (rows0, rows1)
    gsem_b = (gsem0, gsem1)
    dsem_b = (dsem0, dsem1)

    zeros16 = jnp.zeros((16,), jnp.float32)

    def zrow(r, carry):
        for c8 in range(D // 16):
            rows0[r, pl.ds(c8 * 16, 16)] = zeros16
        return carry

    lax.fori_loop(0, ZCH, zrow, 0)

    def zcopy(i, carry):
        z = s + i * NS

        @pl.when(z < NZCH)
        def _():
            pltpu.sync_copy(rows0, acc.at[pl.ds(z * ZCH, ZCH)])

        return carry

    lax.fori_loop(0, ZITER, zcopy, 0)

    pltpu.sync_copy(src_hbm.at[pl.ds(ebase, EPT)], src_all)
    pltpu.sync_copy(ew_hbm.at[pl.ds(ebase, EPT)], ew_all)
    plsc.subcore_barrier()

    def issue(j, b):
        pltpu.async_copy(dst_hbm.at[pl.ds(ebase + j * K, K)], dst_b[b],
                         dsem_b[b])
        pltpu.async_copy(x_hbm.at[src_all.at[pl.ds(j * K, K)]], rows_b[b],
                         gsem_b[b])

    def do_chunk(j, b):
        nxt = j + 1

        @pl.when(nxt < CHUNKS)
        def _():
            issue(nxt, 1 - b)

        pltpu.make_async_copy(dst_hbm.at[pl.ds(0, K)], dst_b[b],
                              dsem_b[b]).wait()
        pltpu.make_async_copy(x_hbm.at[pl.ds(0, K)], rows_b[b],
                              gsem_b[b]).wait()

        rows = rows_b[b]

        def egroup(g, icarry):
            w16 = ew_all[pl.ds(j * K + g * 16, 16)]
            for l in range(16):
                w = lax.gather(w16, jnp.full((16, 1), l, jnp.int32),
                               dimension_numbers=_GDN, slice_sizes=(1,),
                               mode=lax.GatherScatterMode.PROMISE_IN_BOUNDS)
                k = g * 16 + l
                for c8 in range(D // 16):
                    sl = pl.ds(c8 * 16, 16)
                    rows[k, sl] = rows[k, sl] * w
            return icarry

        lax.fori_loop(0, K // 16, egroup, 0)
        pltpu.sync_copy(rows, acc.at[dst_b[b]], add=True)

    issue(0, 0)

    def pair(t, carry):
        do_chunk(2 * t, 0)
        do_chunk(2 * t + 1, 1)
        return carry

    lax.fori_loop(0, CHUNKS // 2, pair, 0)
    do_chunk(CHUNKS - 1, 0)
    plsc.subcore_barrier()

    def ocopy(i, carry):
        z = s + i * NS

        @pl.when(z < NZCH)
        def _():
            base = z * ZCH
            pltpu.sync_copy(acc.at[pl.ds(base, ZCH)],
                            out_hbm.at[c, pl.ds(base, ZCH)])

        return carry

    lax.fori_loop(0, ZITER, ocopy, 0)


BLK = 1000


def _tc_body(x_ref, p0_ref, p1_ref, ws_ref, wn_ref, bs_ref, bn_ref, o_ref):
    neigh = p0_ref[...] + p1_ref[...]
    dn = (((1,), (1,)), ((), ()))
    out = (lax.dot_general(x_ref[...], ws_ref[...], dn,
                           preferred_element_type=jnp.float32)
           + lax.dot_general(neigh, wn_ref[...], dn,
                             preferred_element_type=jnp.float32)
           + bs_ref[...] + bn_ref[...])
    o_ref[...] = jnp.maximum(out, 0.0)


@jax.jit
def _dense_tail(x, p0, p1, W_self, b_self, W_neigh, b_neigh):
    return pl.pallas_call(
        _tc_body,
        grid=(N // BLK,),
        in_specs=[
            pl.BlockSpec((BLK, D), lambda i: (i, 0)),
            pl.BlockSpec((BLK, D), lambda i: (i, 0)),
            pl.BlockSpec((BLK, D), lambda i: (i, 0)),
            pl.BlockSpec((D, D), lambda i: (0, 0)),
            pl.BlockSpec((D, D), lambda i: (0, 0)),
            pl.BlockSpec((1, D), lambda i: (0, 0)),
            pl.BlockSpec((1, D), lambda i: (0, 0)),
        ],
        out_specs=pl.BlockSpec((BLK, D), lambda i: (i, 0)),
        out_shape=jax.ShapeDtypeStruct((N, D), jnp.float32),
    )(x, p0, p1, W_self, W_neigh,
      b_self.reshape(1, D), b_neigh.reshape(1, D))


def kernel(x, edge_index, edge_weight, W_self, b_self, W_neigh, b_neigh):
    dst = edge_index[0].astype(jnp.int32)
    src = edge_index[1].astype(jnp.int32)
    partials = _sc_segsum(x, src, dst, edge_weight)
    return _dense_tail(x, partials[0], partials[1],
                       W_self, b_self, W_neigh, b_neigh)


# speedup vs baseline: 12.0857x; 1.1209x over previous
import functools

import jax
import jax.numpy as jnp
from jax import lax
from jax.experimental import pallas as pl
from jax.experimental.pallas import tpu as pltpu
from jax.experimental.pallas import tpu_sc as plsc

N = 10000
E = 320000
D = 128

NC = 2
NS = 16
NW = NC * NS
EPT = E // NW
K = 80
CHUNKS = EPT // K
ZCH = 80
NZCH = N // ZCH
ZITER = (NZCH + NS - 1) // NS

_mesh = plsc.VectorSubcoreMesh(core_axis_name="c", subcore_axis_name="s")

_GDN = lax.GatherDimensionNumbers(
    offset_dims=(), collapsed_slice_dims=(0,), start_index_map=(0,))


@functools.partial(
    pl.kernel,
    mesh=_mesh,
    out_type=jax.ShapeDtypeStruct((NC, N, D), jnp.float32),
    scratch_types=[
        pltpu.VMEM_SHARED((N, D), jnp.float32),
        pltpu.VMEM((EPT,), jnp.int32),
        pltpu.VMEM((K,), jnp.int32),
        pltpu.VMEM((K,), jnp.int32),
        pltpu.VMEM((K,), jnp.int32),
        pltpu.VMEM((K,), jnp.float32),
        pltpu.VMEM((K,), jnp.float32),
        pltpu.VMEM((K,), jnp.float32),
        pltpu.VMEM((K, D), jnp.float32),
        pltpu.VMEM((K, D), jnp.float32),
        pltpu.VMEM((K, D), jnp.float32),
        pltpu.SemaphoreType.DMA,
        pltpu.SemaphoreType.DMA,
        pltpu.SemaphoreType.DMA,
        pltpu.SemaphoreType.DMA,
        pltpu.SemaphoreType.DMA,
        pltpu.SemaphoreType.DMA,
        pltpu.SemaphoreType.DMA,
        pltpu.SemaphoreType.DMA,
        pltpu.SemaphoreType.DMA,
    ],
)
def _sc_segsum(x_hbm, src_hbm, dst_hbm, ew_hbm, out_hbm,
               acc, src_all, dst0, dst1, dst2, ew0, ew1, ew2,
               rows0, rows1, rows2,
               gsem0, gsem1, gsem2, dsem0, dsem1, dsem2,
               ssem0, ssem1, ssem2):
    c = lax.axis_index("c")
    s = lax.axis_index("s")
    wid = c * NS + s
    ebase = wid * EPT

    dst_b = (dst0, dst1, dst2)
    ew_b = (ew0, ew1, ew2)
    rows_b = (rows0, rows1, rows2)
    gsem_b = (gsem0, gsem1, gsem2)
    dsem_b = (dsem0, dsem1, dsem2)
    ssem_b = (ssem0, ssem1, ssem2)

    zeros16 = jnp.zeros((16,), jnp.float32)

    def zrow(r, carry):
        for c8 in range(D // 16):
            rows0[r, pl.ds(c8 * 16, 16)] = zeros16
        return carry

    lax.fori_loop(0, ZCH, zrow, 0)

    def zcopy(i, carry):
        z = s + i * NS

        @pl.when(z < NZCH)
        def _():
            pltpu.sync_copy(rows0, acc.at[pl.ds(z * ZCH, ZCH)])

        return carry

    lax.fori_loop(0, ZITER, zcopy, 0)

    pltpu.sync_copy(src_hbm.at[pl.ds(ebase, EPT)], src_all)
    plsc.subcore_barrier()

    def issue(j, b):
        pltpu.async_copy(dst_hbm.at[pl.ds(ebase + j * K, K)], dst_b[b],
                         dsem_b[b])
        pltpu.async_copy(ew_hbm.at[pl.ds(ebase + j * K, K)], ew_b[b],
                         dsem_b[b])
        pltpu.async_copy(x_hbm.at[src_all.at[pl.ds(j * K, K)]], rows_b[b],
                         gsem_b[b])

    def wait_scatter(b):
        pltpu.make_async_copy(x_hbm.at[pl.ds(0, K)], rows_b[b],
                              ssem_b[b]).wait()

    def do_chunk(j, b, first=False):
        pltpu.make_async_copy(dst_hbm.at[pl.ds(0, K)], dst_b[b],
                              dsem_b[b]).wait()
        pltpu.make_async_copy(ew_hbm.at[pl.ds(0, K)], ew_b[b],
                              dsem_b[b]).wait()
        pltpu.make_async_copy(x_hbm.at[pl.ds(0, K)], rows_b[b],
                              gsem_b[b]).wait()

        rows = rows_b[b]
        ew = ew_b[b]

        def egroup(g, icarry):
            w16 = ew[pl.ds(g * 16, 16)]
            for l in range(16):
                w = lax.gather(w16, jnp.full((16, 1), l, jnp.int32),
                               dimension_numbers=_GDN, slice_sizes=(1,),
                               mode=lax.GatherScatterMode.PROMISE_IN_BOUNDS)
                k = g * 16 + l
                for c8 in range(D // 16):
                    sl = pl.ds(c8 * 16, 16)
                    rows[k, sl] = rows[k, sl] * w
            return icarry

        lax.fori_loop(0, K // 16, egroup, 0)
        pltpu.async_copy(rows, acc.at[dst_b[b]], ssem_b[b], add=True)

        nxt = j + 2
        bn = (b + 2) % 3

        @pl.when(nxt < CHUNKS)
        def _():
            if not first:
                wait_scatter(bn)
            issue(nxt, bn)

    issue(0, 0)
    issue(1, 1)
    do_chunk(0, 0, first=True)

    def triple(t, carry):
        j = 3 * t + 1
        do_chunk(j, 1)
        do_chunk(j + 1, 2)
        do_chunk(j + 2, 0)
        return carry

    lax.fori_loop(0, (CHUNKS - 2) // 3, triple, 0)
    do_chunk(CHUNKS - 1, 1)
    wait_scatter(0)
    wait_scatter(1)
    wait_scatter(2)
    plsc.subcore_barrier()

    def ocopy(i, carry):
        z = s + i * NS

        @pl.when(z < NZCH)
        def _():
            base = z * ZCH
            pltpu.sync_copy(acc.at[pl.ds(base, ZCH)],
                            out_hbm.at[c, pl.ds(base, ZCH)])

        return carry

    lax.fori_loop(0, ZITER, ocopy, 0)


BLK = 1000


def _tc_body(x_ref, p0_ref, p1_ref, ws_ref, wn_ref, bs_ref, bn_ref, o_ref):
    neigh = p0_ref[...] + p1_ref[...]
    dn = (((1,), (1,)), ((), ()))
    out = (lax.dot_general(x_ref[...], ws_ref[...], dn,
                           preferred_element_type=jnp.float32)
           + lax.dot_general(neigh, wn_ref[...], dn,
                             preferred_element_type=jnp.float32)
           + bs_ref[...] + bn_ref[...])
    o_ref[...] = jnp.maximum(out, 0.0)


@jax.jit
def _dense_tail(x, p0, p1, W_self, b_self, W_neigh, b_neigh):
    return pl.pallas_call(
        _tc_body,
        grid=(N // BLK,),
        in_specs=[
            pl.BlockSpec((BLK, D), lambda i: (i, 0)),
            pl.BlockSpec((BLK, D), lambda i: (i, 0)),
            pl.BlockSpec((BLK, D), lambda i: (i, 0)),
            pl.BlockSpec((D, D), lambda i: (0, 0)),
            pl.BlockSpec((D, D), lambda i: (0, 0)),
            pl.BlockSpec((1, D), lambda i: (0, 0)),
            pl.BlockSpec((1, D), lambda i: (0, 0)),
        ],
        out_specs=pl.BlockSpec((BLK, D), lambda i: (i, 0)),
        out_shape=jax.ShapeDtypeStruct((N, D), jnp.float32),
    )(x, p0, p1, W_self, W_neigh,
      b_self.reshape(1, D), b_neigh.reshape(1, D))


def kernel(x, edge_index, edge_weight, W_self, b_self, W_neigh, b_neigh):
    dst = edge_index[0].astype(jnp.int32)
    src = edge_index[1].astype(jnp.int32)
    partials = _sc_segsum(x, src, dst, edge_weight)
    return _dense_tail(x, partials[0], partials[1],
                       W_self, b_self, W_neigh, b_neigh)


# speedup vs baseline: 12.2040x; 1.0098x over previous
import functools

import jax
import jax.numpy as jnp
from jax import lax
from jax.experimental import pallas as pl
from jax.experimental.pallas import tpu as pltpu
from jax.experimental.pallas import tpu_sc as plsc

N = 10000
E = 320000
D = 128

NC = 2
NS = 16
NW = NC * NS
EPT = E // NW
K = 80
CHUNKS = EPT // K
ZCH = 80
NZCH = N // ZCH
ZITER = (NZCH + NS - 1) // NS

_mesh = plsc.VectorSubcoreMesh(core_axis_name="c", subcore_axis_name="s")

_GDN = lax.GatherDimensionNumbers(
    offset_dims=(), collapsed_slice_dims=(0,), start_index_map=(0,))


@functools.partial(
    pl.kernel,
    mesh=_mesh,
    out_type=jax.ShapeDtypeStruct((NC, N, D), jnp.float32),
    scratch_types=[
        pltpu.VMEM_SHARED((N, D), jnp.float32),
        pltpu.VMEM((EPT,), jnp.int32),
        pltpu.VMEM((K,), jnp.int32),
        pltpu.VMEM((K,), jnp.int32),
        pltpu.VMEM((K,), jnp.int32),
        pltpu.VMEM((K,), jnp.float32),
        pltpu.VMEM((K,), jnp.float32),
        pltpu.VMEM((K,), jnp.float32),
        pltpu.VMEM((K, D), jnp.float32),
        pltpu.VMEM((K, D), jnp.float32),
        pltpu.VMEM((K, D), jnp.float32),
        pltpu.SemaphoreType.DMA,
        pltpu.SemaphoreType.DMA,
        pltpu.SemaphoreType.DMA,
        pltpu.SemaphoreType.DMA,
        pltpu.SemaphoreType.DMA,
        pltpu.SemaphoreType.DMA,
        pltpu.SemaphoreType.DMA,
        pltpu.SemaphoreType.DMA,
        pltpu.SemaphoreType.DMA,
    ],
)
def _sc_segsum(x_hbm, src_hbm, dst_hbm, ew_hbm, out_hbm,
               acc, src_all, dst0, dst1, dst2, ew0, ew1, ew2,
               rows0, rows1, rows2,
               gsem0, gsem1, gsem2, dsem0, dsem1, dsem2,
               ssem0, ssem1, ssem2):
    c = lax.axis_index("c")
    s = lax.axis_index("s")
    wid = c * NS + s
    ebase = wid * EPT

    dst_b = (dst0, dst1, dst2)
    ew_b = (ew0, ew1, ew2)
    rows_b = (rows0, rows1, rows2)
    gsem_b = (gsem0, gsem1, gsem2)
    dsem_b = (dsem0, dsem1, dsem2)
    ssem_b = (ssem0, ssem1, ssem2)

    pltpu.sync_copy(src_hbm.at[pl.ds(ebase, EPT)], src_all)

    def issue(j, b):
        pltpu.async_copy(dst_hbm.at[pl.ds(ebase + j * K, K)], dst_b[b],
                         dsem_b[b])
        pltpu.async_copy(ew_hbm.at[pl.ds(ebase + j * K, K)], ew_b[b],
                         dsem_b[b])
        pltpu.async_copy(x_hbm.at[src_all.at[pl.ds(j * K, K)]], rows_b[b],
                         gsem_b[b])

    def wait_scatter(b):
        pltpu.make_async_copy(x_hbm.at[pl.ds(0, K)], rows_b[b],
                              ssem_b[b]).wait()

    def do_chunk(j, b, first=False):
        pltpu.make_async_copy(dst_hbm.at[pl.ds(0, K)], dst_b[b],
                              dsem_b[b]).wait()
        pltpu.make_async_copy(ew_hbm.at[pl.ds(0, K)], ew_b[b],
                              dsem_b[b]).wait()
        pltpu.make_async_copy(x_hbm.at[pl.ds(0, K)], rows_b[b],
                              gsem_b[b]).wait()

        rows = rows_b[b]
        ew = ew_b[b]

        def egroup(g, icarry):
            w16 = ew[pl.ds(g * 16, 16)]
            for l in range(16):
                w = lax.gather(w16, jnp.full((16, 1), l, jnp.int32),
                               dimension_numbers=_GDN, slice_sizes=(1,),
                               mode=lax.GatherScatterMode.PROMISE_IN_BOUNDS)
                row = rows.at[g * 16 + l]
                for c8 in range(D // 16):
                    sl = pl.ds(c8 * 16, 16)
                    row[sl] = row[sl] * w
            return icarry

        lax.fori_loop(0, K // 16, egroup, 0)
        pltpu.async_copy(rows, acc.at[dst_b[b]], ssem_b[b], add=True)

        nxt = j + 2
        bn = (b + 2) % 3

        @pl.when(nxt < CHUNKS)
        def _():
            if not first:
                wait_scatter(bn)
            issue(nxt, bn)

    issue(0, 0)
    issue(1, 1)

    zeros16 = jnp.zeros((16,), jnp.float32)

    def zrow(r, carry):
        for c8 in range(D // 16):
            rows2[r, pl.ds(c8 * 16, 16)] = zeros16
        return carry

    lax.fori_loop(0, ZCH, zrow, 0)

    def zcopy(i, carry):
        z = s + i * NS

        @pl.when(z < NZCH)
        def _():
            pltpu.sync_copy(rows2, acc.at[pl.ds(z * ZCH, ZCH)])

        return carry

    lax.fori_loop(0, ZITER, zcopy, 0)
    plsc.subcore_barrier()

    do_chunk(0, 0, first=True)

    def triple(t, carry):
        j = 3 * t + 1
        do_chunk(j, 1)
        do_chunk(j + 1, 2)
        do_chunk(j + 2, 0)
        return carry

    lax.fori_loop(0, (CHUNKS - 2) // 3, triple, 0)
    do_chunk(CHUNKS - 1, 1)
    wait_scatter(0)
    wait_scatter(1)
    wait_scatter(2)
    plsc.subcore_barrier()

    def ocopy(i, carry):
        z = s + i * NS

        @pl.when(z < NZCH)
        def _():
            base = z * ZCH
            pltpu.sync_copy(acc.at[pl.ds(base, ZCH)],
                            out_hbm.at[c, pl.ds(base, ZCH)])

        return carry

    lax.fori_loop(0, ZITER, ocopy, 0)


BLK = 1000


def _tc_body(x_ref, p0_ref, p1_ref, ws_ref, wn_ref, bs_ref, bn_ref, o_ref):
    neigh = p0_ref[...] + p1_ref[...]
    dn = (((1,), (1,)), ((), ()))
    out = (lax.dot_general(x_ref[...], ws_ref[...], dn,
                           preferred_element_type=jnp.float32)
           + lax.dot_general(neigh, wn_ref[...], dn,
                             preferred_element_type=jnp.float32)
           + bs_ref[...] + bn_ref[...])
    o_ref[...] = jnp.maximum(out, 0.0)


@jax.jit
def _dense_tail(x, p0, p1, W_self, b_self, W_neigh, b_neigh):
    return pl.pallas_call(
        _tc_body,
        grid=(N // BLK,),
        in_specs=[
            pl.BlockSpec((BLK, D), lambda i: (i, 0)),
            pl.BlockSpec((BLK, D), lambda i: (i, 0)),
            pl.BlockSpec((BLK, D), lambda i: (i, 0)),
            pl.BlockSpec((D, D), lambda i: (0, 0)),
            pl.BlockSpec((D, D), lambda i: (0, 0)),
            pl.BlockSpec((1, D), lambda i: (0, 0)),
            pl.BlockSpec((1, D), lambda i: (0, 0)),
        ],
        out_specs=pl.BlockSpec((BLK, D), lambda i: (i, 0)),
        out_shape=jax.ShapeDtypeStruct((N, D), jnp.float32),
    )(x, p0, p1, W_self, W_neigh,
      b_self.reshape(1, D), b_neigh.reshape(1, D))


def kernel(x, edge_index, edge_weight, W_self, b_self, W_neigh, b_neigh):
    dst = edge_index[0].astype(jnp.int32)
    src = edge_index[1].astype(jnp.int32)
    partials = _sc_segsum(x, src, dst, edge_weight)
    return _dense_tail(x, partials[0], partials[1],
                       W_self, b_self, W_neigh, b_neigh)
